# baseline (speedup 1.0000x reference)
import functools

import jax
import jax.numpy as jnp
from jax import lax
from jax.experimental import pallas as pl
from jax.experimental.pallas import tpu as pltpu
from jax.experimental.pallas import tpu_sc as plsc

_N = 10000
_E = 320000
_G = 64
_MAXD = 10
_D = 128
_L = 3

_NC = 2
_NS = 16
_EPT = _E // (_NC * _NS)
_CHUNK = 80
_NCHUNK = _EPT // _CHUNK
_NV = _CHUNK // 16
_HC = 96
_HEPT = 10272
_HNCH = _HEPT // _HC
_NHROW = _N + 8
_RPT = 624
_NDR = 640
_DRT = _NDR // _NS


def _sc_mesh():
  return plsc.VectorSubcoreMesh(core_axis_name="c", subcore_axis_name="s",
                                num_cores=_NC)


def _stripe_copy(src, dst, buf, rows, base, bufrows=_CHUNK):
  done = 0
  while done < rows:
    n = min(bufrows, rows - done)
    r = pl.multiple_of(base + done, 8)
    pltpu.sync_copy(src.at[pl.ds(r, n)], buf.at[pl.ds(0, n)])
    pltpu.sync_copy(buf.at[pl.ds(0, n)], dst.at[pl.ds(r, n)])
    done += n


def _stripe_zero(z_hbm, dst, buf, rows, base, bufrows=_CHUNK):
  pltpu.sync_copy(z_hbm.at[pl.ds(0, bufrows)], buf)
  done = 0
  while done < rows:
    n = min(bufrows, rows - done)
    r = pl.multiple_of(base + done, 8)
    pltpu.sync_copy(buf.at[pl.ds(0, n)], dst.at[pl.ds(r, n)])
    done += n


@functools.partial(
    pl.kernel,
    out_type=jax.ShapeDtypeStruct((_NC, _NDR, _D), jnp.float32),
    mesh=_sc_mesh(),
    scratch_types=(
        pltpu.VMEM_SHARED((_NDR, _D), jnp.float32),
        (pltpu.VMEM((_CHUNK,), jnp.int32),) * 2,
        (pltpu.VMEM((_CHUNK,), jnp.int32),) * 2,
        (pltpu.VMEM((_CHUNK, _D), jnp.float32),) * 2,
        (pltpu.SemaphoreType.DMA,) * 2,
        (pltpu.SemaphoreType.DMA,) * 2,
        (pltpu.SemaphoreType.DMA,) * 2,
    ),
)
def _sc_deg(dst_hbm, eye_hbm, z128_hbm, deg_out, shared_deg, dhi_v,
            low_v, oneh_v, gsem, ssem, isem):
  cid = lax.axis_index("c")
  sid = lax.axis_index("s")
  _stripe_zero(z128_hbm, shared_deg, oneh_v[0], _DRT, sid * _DRT)
  plsc.subcore_barrier()
  ebase = (cid * _NS + sid) * _EPT
  lanes = lax.iota(jnp.int32, 16)
  spread = lax.shift_left(lax.bitwise_and(lanes, 7), 4)

  def idx_start(c, b):
    off = pl.multiple_of(ebase + c * _CHUNK, 8)
    pltpu.async_copy(dst_hbm.at[pl.ds(off, _CHUNK)], dhi_v[b], isem[b])

  def idx_wait(b):
    pltpu.make_async_copy(dst_hbm.at[pl.ds(0, _CHUNK)], dhi_v[b],
                          isem[b]).wait()
    for k in range(_NV):
      dv = dhi_v[b][pl.ds(16 * k, 16)]
      low_v[b][pl.ds(16 * k, 16)] = lax.bitwise_or(
          lax.bitwise_and(dv, 15), spread)
      dhi_v[b][pl.ds(16 * k, 16)] = lax.shift_right_logical(dv, 4)

  def gather_start(b):
    pltpu.async_copy(eye_hbm.at[low_v[b]], oneh_v[b], gsem[b])

  def gather_wait(b):
    pltpu.make_async_copy(eye_hbm.at[low_v[b]], oneh_v[b], gsem[b]).wait()

  def scat_start(b):
    pltpu.async_copy(oneh_v[b], shared_deg.at[dhi_v[b]], ssem[b], add=True)

  def scat_wait(b):
    pltpu.make_async_copy(oneh_v[b], shared_deg.at[dhi_v[b]],
                          ssem[b]).wait()

  idx_start(0, 0)
  idx_wait(0)
  gather_start(0)
  idx_start(1, 1)

  def body(t, carry):
    idx_wait(1)
    gather_start(1)
    gather_wait(0)
    scat_start(0)
    scat_wait(0)

    @pl.when(2 * t + 2 < _NCHUNK)
    def _():
      idx_start(2 * t + 2, 0)
    gather_wait(1)
    scat_start(1)

    @pl.when(2 * t + 2 < _NCHUNK)
    def _():
      idx_wait(0)
      gather_start(0)
    scat_wait(1)

    @pl.when(2 * t + 3 < _NCHUNK)
    def _():
      idx_start(2 * t + 3, 1)
    return carry

  lax.fori_loop(0, (_NCHUNK - 1) // 2, body, 0)
  gather_wait(0)
  scat_start(0)
  scat_wait(0)
  plsc.subcore_barrier()
  _stripe_copy(shared_deg, deg_out.at[cid], oneh_v[0], _DRT, sid * _DRT)


@functools.partial(
    pl.kernel,
    out_type=jax.ShapeDtypeStruct((_NC, _N, _D), jnp.float32),
    mesh=_sc_mesh(),
    scratch_types=(
        pltpu.VMEM_SHARED((_NHROW, _D), jnp.float32),
        (pltpu.VMEM((_HC,), jnp.int32),) * 2,
        (pltpu.VMEM((_HC,), jnp.int32),) * 2,
        (pltpu.VMEM((_HC, _D), jnp.float32),) * 2,
        (pltpu.SemaphoreType.DMA,) * 2,
        (pltpu.SemaphoreType.DMA,) * 2,
        (pltpu.SemaphoreType.DMA,) * 2,
    ),
)
def _sc_agg(x_hbm, src_hbm, dst_hbm, z128_hbm, h_out, shared_h, src_v,
            dst_v, rows_v, gsem, ssem, isem):
  cid = lax.axis_index("c")
  sid = lax.axis_index("s")
  _stripe_zero(z128_hbm, shared_h, rows_v[0], _RPT, sid * _RPT,
               bufrows=_HC)

  @pl.when(sid == _NS - 1)
  def _():
    _stripe_zero(z128_hbm, shared_h, rows_v[0], _NHROW - _NS * _RPT,
                 _NS * _RPT, bufrows=_HC)

  plsc.subcore_barrier()
  ebase = (cid * _NS + sid) * _HEPT

  def idx_start(c, b):
    off = pl.multiple_of(ebase + c * _HC, 8)
    pltpu.async_copy(src_hbm.at[pl.ds(off, _HC)], src_v[b], isem[b])
    pltpu.async_copy(dst_hbm.at[pl.ds(off, _HC)], dst_v[b], isem[b])

  def idx_wait(b):
    pltpu.make_async_copy(src_hbm.at[pl.ds(0, _HC)], src_v[b],
                          isem[b]).wait()
    pltpu.make_async_copy(dst_hbm.at[pl.ds(0, _HC)], dst_v[b],
                          isem[b]).wait()

  def gather_start(b):
    pltpu.async_copy(x_hbm.at[src_v[b]], rows_v[b], gsem[b])

  def gather_wait(b):
    pltpu.make_async_copy(x_hbm.at[src_v[b]], rows_v[b], gsem[b]).wait()

  def scat_start(b):
    pltpu.async_copy(rows_v[b], shared_h.at[dst_v[b]], ssem[b], add=True)

  def scat_wait(b):
    pltpu.make_async_copy(rows_v[b], shared_h.at[dst_v[b]], ssem[b]).wait()

  idx_start(0, 0)
  idx_wait(0)
  gather_start(0)
  idx_start(1, 1)

  def body(t, carry):
    idx_wait(1)
    gather_start(1)
    gather_wait(0)
    scat_start(0)
    scat_wait(0)

    @pl.when(2 * t + 2 < _HNCH)
    def _():
      idx_start(2 * t + 2, 0)
    gather_wait(1)
    scat_start(1)

    @pl.when(2 * t + 2 < _HNCH)
    def _():
      idx_wait(0)
      gather_start(0)
    scat_wait(1)

    @pl.when(2 * t + 3 < _HNCH)
    def _():
      idx_start(2 * t + 3, 1)
    return carry

  lax.fori_loop(0, (_HNCH - 1) // 2, body, 0)
  gather_wait(0)
  scat_start(0)
  scat_wait(0)
  plsc.subcore_barrier()
  _stripe_copy(shared_h, h_out.at[cid], rows_v[0], _RPT, sid * _RPT,
               bufrows=_HC)

  @pl.when(sid == _NS - 1)
  def _():
    _stripe_copy(shared_h, h_out.at[cid], rows_v[0], _N - _NS * _RPT,
                 _NS * _RPT, bufrows=_HC)


_NB = 10
_B = _N // _NB


def _tc_body(x_ref, h_ref, deg_ref, wcat_ref, bl_ref, linwt_ref,
             batch_ref, acc_ref, xo_ref, pool_ref):
  i = pl.program_id(0)
  x = x_ref[...]
  h = h_ref[0] + h_ref[1]
  degf = jnp.minimum(deg_ref[0] + deg_ref[1], float(_MAXD))
  hx = jnp.concatenate([h, x], axis=1)
  out = jnp.zeros((_B, _D), jnp.float32)
  for d in range(_MAXD + 1):
    r = jnp.dot(hx, wcat_ref[d], preferred_element_type=jnp.float32)
    r = r + bl_ref[d][None, :]
    out = jnp.where(degf == float(d), r, out)
  xo = jax.nn.sigmoid(out)
  xo_ref[...] = xo
  y = jnp.dot(xo, linwt_ref[...], preferred_element_type=jnp.float32)
  onehot = (batch_ref[...] == lax.broadcasted_iota(jnp.int32, (1, _G), 1)
            ).astype(jnp.float32)
  p = lax.dot_general(onehot, y, (((0,), (0,)), ((), ())),
                      preferred_element_type=jnp.float32)

  @pl.when(i == 0)
  def _():
    pool_ref[...] = acc_ref[...]

  pool_ref[...] += p


_tc_layer = pl.pallas_call(
    _tc_body,
    grid=(_NB,),
    in_specs=[
        pl.BlockSpec((_B, _D), lambda i: (i, 0)),
        pl.BlockSpec((_NC, _B, _D), lambda i: (0, i, 0)),
        pl.BlockSpec((_NC, _B, 1), lambda i: (0, i, 0)),
        pl.BlockSpec((_MAXD + 1, 2 * _D, _D), lambda i: (0, 0, 0)),
        pl.BlockSpec((16, _D), lambda i: (0, 0)),
        pl.BlockSpec((_D, _D), lambda i: (0, 0)),
        pl.BlockSpec((_B, 1), lambda i: (i, 0)),
        pl.BlockSpec((_G, _D), lambda i: (0, 0)),
    ],
    out_specs=[
        pl.BlockSpec((_B, _D), lambda i: (i, 0)),
        pl.BlockSpec((_G, _D), lambda i: (0, 0)),
    ],
    out_shape=[
        jax.ShapeDtypeStruct((_N, _D), jnp.float32),
        jax.ShapeDtypeStruct((_G, _D), jnp.float32),
    ],
)


def _mlp_body(acc_ref, mlpwt_ref, mlpb_ref, out_ref):
  out_ref[...] = (jnp.dot(acc_ref[...], mlpwt_ref[...],
                          preferred_element_type=jnp.float32)
                  + mlpb_ref[...])


_tc_mlp = pl.pallas_call(
    _mlp_body,
    out_shape=jax.ShapeDtypeStruct((_G, _D), jnp.float32),
)


@jax.jit
def kernel(x, edge_index, batch, Wl, bl, Wr, lin_W, mlp_W, mlp_b):
  src = edge_index[0]
  dst = edge_index[1]
  pad = _HEPT - _EPT
  src_p = jnp.pad(src.reshape(_NC * _NS, _EPT), ((0, 0), (0, pad))).ravel()
  dst_p = jnp.pad(dst.reshape(_NC * _NS, _EPT), ((0, 0), (0, pad)),
                  constant_values=_N).ravel()
  z128 = jnp.zeros((_HC, _D), jnp.float32)
  wcat = jnp.concatenate(
      [jnp.transpose(Wl, (0, 1, 3, 2)), jnp.transpose(Wr, (0, 1, 3, 2))],
      axis=2)
  blp = jnp.pad(bl, ((0, 0), (0, 16 - (_MAXD + 1)), (0, 0)))
  linwt = jnp.transpose(lin_W, (0, 2, 1))
  mlpwt = mlp_W.T
  mlpb2 = mlp_b[None, :]
  batch2 = batch[:, None]
  acc = jnp.zeros((_G, _D), jnp.float32)

  eye = jnp.tile(jnp.eye(16, _D, dtype=jnp.float32), (8, 1))
  deg_plane = _sc_deg(dst, eye, z128)
  deg2 = deg_plane[:, :, :16].reshape(_NC, _NDR * 16, 1)[:, :_N]

  def layer(carry, ws):
    xc, accc = carry
    wcat_l, bl_l, linwt_l = ws
    h2 = _sc_agg(xc, src_p, dst_p, z128)
    xn, accn = _tc_layer(xc, h2, deg2, wcat_l, bl_l, linwt_l, batch2, accc)
    return (xn, accn), None

  (x, acc), _ = lax.scan(layer, (x, acc), (wcat, blp, linwt))
  return _tc_mlp(acc, mlpwt, mlpb2)


# speedup vs baseline: 2.1808x; 2.1808x over previous
import functools

import jax
import jax.numpy as jnp
from jax import lax
from jax.experimental import pallas as pl
from jax.experimental.pallas import tpu as pltpu
from jax.experimental.pallas import tpu_sc as plsc

_N = 10000
_E = 320000
_G = 64
_MAXD = 10
_D = 128
_L = 3

_NC = 2
_NS = 16
_EPT = _E // (_NC * _NS)
_CHUNK = 80
_NCHUNK = _EPT // _CHUNK
_NV = _CHUNK // 16
_RPT = 624
_NDR = 640
_DRT = _NDR // _NS


def _sc_mesh():
  return plsc.VectorSubcoreMesh(core_axis_name="c", subcore_axis_name="s",
                                num_cores=_NC)


def _stripe_copy(src, dst, buf, rows, base):
  done = 0
  while done < rows:
    n = min(_CHUNK, rows - done)
    r = pl.multiple_of(base + done, 8)
    pltpu.sync_copy(src.at[pl.ds(r, n)], buf.at[pl.ds(0, n)])
    pltpu.sync_copy(buf.at[pl.ds(0, n)], dst.at[pl.ds(r, n)])
    done += n


def _stripe_zero(z_hbm, dst, buf, rows, base):
  pltpu.sync_copy(z_hbm, buf)
  done = 0
  while done < rows:
    n = min(_CHUNK, rows - done)
    r = pl.multiple_of(base + done, 8)
    pltpu.sync_copy(buf.at[pl.ds(0, n)], dst.at[pl.ds(r, n)])
    done += n


@functools.partial(
    pl.kernel,
    out_type=jax.ShapeDtypeStruct((_NC, _NDR, _D), jnp.float32),
    mesh=_sc_mesh(),
    scratch_types=(
        pltpu.VMEM_SHARED((_NDR, _D), jnp.float32),
        (pltpu.VMEM((_CHUNK,), jnp.int32),) * 2,
        (pltpu.VMEM((_CHUNK,), jnp.int32),) * 2,
        (pltpu.VMEM((_CHUNK,), jnp.int32),) * 2,
        (pltpu.VMEM((_CHUNK, _D), jnp.float32),) * 2,
        (pltpu.SemaphoreType.DMA,) * 2,
        (pltpu.SemaphoreType.DMA,) * 2,
        (pltpu.SemaphoreType.DMA,) * 2,
    ),
)
def _sc_deg(dst_hbm, eye_hbm, z128_hbm, deg_out, shared_deg, dst_v, dhi_v,
            low_v, oneh_v, gsem, ssem, isem):
  cid = lax.axis_index("c")
  sid = lax.axis_index("s")
  _stripe_zero(z128_hbm, shared_deg, oneh_v[0], _DRT, sid * _DRT)
  plsc.subcore_barrier()
  ebase = (cid * _NS + sid) * _EPT
  lanes = lax.iota(jnp.int32, 16)
  spread = lax.shift_left(lax.bitwise_and(lanes, 7), 4)

  def idx_start(c, b):
    off = pl.multiple_of(ebase + c * _CHUNK, 8)
    pltpu.async_copy(dst_hbm.at[pl.ds(off, _CHUNK)], dst_v[b], isem[b])

  def idx_wait(b):
    pltpu.make_async_copy(dst_hbm.at[pl.ds(0, _CHUNK)], dst_v[b],
                          isem[b]).wait()
    for k in range(_NV):
      dv = dst_v[b][pl.ds(16 * k, 16)]
      dhi_v[b][pl.ds(16 * k, 16)] = lax.shift_right_logical(dv, 4)
      low_v[b][pl.ds(16 * k, 16)] = lax.bitwise_or(
          lax.bitwise_and(dv, 15), spread)

  def gather_start(b):
    pltpu.async_copy(eye_hbm.at[low_v[b]], oneh_v[b], gsem[b])

  def gather_wait(b):
    pltpu.make_async_copy(eye_hbm.at[low_v[b]], oneh_v[b], gsem[b]).wait()

  def scat_start(b):
    pltpu.async_copy(oneh_v[b], shared_deg.at[dhi_v[b]], ssem[b], add=True)

  def scat_wait(b):
    pltpu.make_async_copy(oneh_v[b], shared_deg.at[dhi_v[b]],
                          ssem[b]).wait()

  idx_start(0, 0)
  idx_wait(0)
  gather_start(0)
  idx_start(1, 1)

  def body(t, carry):
    idx_wait(1)
    gather_start(1)
    gather_wait(0)
    scat_start(0)
    scat_wait(0)

    @pl.when(2 * t + 2 < _NCHUNK)
    def _():
      idx_start(2 * t + 2, 0)
    gather_wait(1)
    scat_start(1)

    @pl.when(2 * t + 2 < _NCHUNK)
    def _():
      idx_wait(0)
      gather_start(0)
    scat_wait(1)

    @pl.when(2 * t + 3 < _NCHUNK)
    def _():
      idx_start(2 * t + 3, 1)
    return carry

  lax.fori_loop(0, (_NCHUNK - 1) // 2, body, 0)
  gather_wait(0)
  scat_start(0)
  scat_wait(0)
  plsc.subcore_barrier()
  _stripe_copy(shared_deg, deg_out.at[cid], oneh_v[0], _DRT, sid * _DRT)


@functools.partial(
    pl.kernel,
    out_type=jax.ShapeDtypeStruct((_NC, _N, _D), jnp.float32),
    mesh=_sc_mesh(),
    scratch_types=(
        pltpu.VMEM_SHARED((_N, _D), jnp.float32),
        (pltpu.VMEM((_CHUNK,), jnp.int32),) * 2,
        (pltpu.VMEM((_CHUNK,), jnp.int32),) * 2,
        (pltpu.VMEM((_CHUNK, _D), jnp.float32),) * 2,
        (pltpu.SemaphoreType.DMA,) * 2,
        (pltpu.SemaphoreType.DMA,) * 2,
        (pltpu.SemaphoreType.DMA,) * 2,
    ),
)
def _sc_agg(x_hbm, src_hbm, dst_hbm, z128_hbm, h_out, shared_h, src_v,
            dst_v, rows_v, gsem, ssem, isem):
  cid = lax.axis_index("c")
  sid = lax.axis_index("s")
  _stripe_zero(z128_hbm, shared_h, rows_v[0], _RPT, sid * _RPT)

  @pl.when(sid == _NS - 1)
  def _():
    _stripe_zero(z128_hbm, shared_h, rows_v[0], _N - _NS * _RPT,
                 _NS * _RPT)

  plsc.subcore_barrier()
  ebase = (cid * _NS + sid) * _EPT

  def idx_start(c, b):
    off = pl.multiple_of(ebase + c * _CHUNK, 8)
    pltpu.async_copy(src_hbm.at[pl.ds(off, _CHUNK)], src_v[b], isem[b])
    pltpu.async_copy(dst_hbm.at[pl.ds(off, _CHUNK)], dst_v[b], isem[b])

  def idx_wait(b):
    pltpu.make_async_copy(src_hbm.at[pl.ds(0, _CHUNK)], src_v[b],
                          isem[b]).wait()
    pltpu.make_async_copy(dst_hbm.at[pl.ds(0, _CHUNK)], dst_v[b],
                          isem[b]).wait()

  def gather_start(b):
    pltpu.async_copy(x_hbm.at[src_v[b]], rows_v[b], gsem[b])

  def gather_wait(b):
    pltpu.make_async_copy(x_hbm.at[src_v[b]], rows_v[b], gsem[b]).wait()

  def scat_start(b):
    pltpu.async_copy(rows_v[b], shared_h.at[dst_v[b]], ssem[b], add=True)

  def scat_wait(b):
    pltpu.make_async_copy(rows_v[b], shared_h.at[dst_v[b]], ssem[b]).wait()

  idx_start(0, 0)
  idx_wait(0)
  gather_start(0)
  idx_start(1, 1)

  def body(t, carry):
    idx_wait(1)
    gather_start(1)
    gather_wait(0)
    scat_start(0)
    scat_wait(0)

    @pl.when(2 * t + 2 < _NCHUNK)
    def _():
      idx_start(2 * t + 2, 0)
    gather_wait(1)
    scat_start(1)

    @pl.when(2 * t + 2 < _NCHUNK)
    def _():
      idx_wait(0)
      gather_start(0)
    scat_wait(1)

    @pl.when(2 * t + 3 < _NCHUNK)
    def _():
      idx_start(2 * t + 3, 1)
    return carry

  lax.fori_loop(0, (_NCHUNK - 1) // 2, body, 0)
  gather_wait(0)
  scat_start(0)
  scat_wait(0)
  plsc.subcore_barrier()
  _stripe_copy(shared_h, h_out.at[cid], rows_v[0], _RPT, sid * _RPT)

  @pl.when(sid == _NS - 1)
  def _():
    _stripe_copy(shared_h, h_out.at[cid], rows_v[0], _N - _NS * _RPT,
                 _NS * _RPT)


_NB = 10
_B = _N // _NB


def _tc_body(x_ref, h_ref, deg_ref, wcat_ref, bl_ref, linwt_ref,
             batch_ref, acc_ref, xo_ref, pool_ref):
  i = pl.program_id(0)
  x = x_ref[...]
  h = h_ref[0] + h_ref[1]
  degf = jnp.minimum(deg_ref[0] + deg_ref[1], float(_MAXD))
  hx = jnp.concatenate([h, x], axis=1)
  out = jnp.zeros((_B, _D), jnp.float32)
  for d in range(_MAXD + 1):
    r = jnp.dot(hx, wcat_ref[d], preferred_element_type=jnp.float32)
    r = r + bl_ref[d][None, :]
    out = jnp.where(degf == float(d), r, out)
  xo = jax.nn.sigmoid(out)
  xo_ref[...] = xo
  y = jnp.dot(xo, linwt_ref[...], preferred_element_type=jnp.float32)
  onehot = (batch_ref[...] == lax.broadcasted_iota(jnp.int32, (1, _G), 1)
            ).astype(jnp.float32)
  p = lax.dot_general(onehot, y, (((0,), (0,)), ((), ())),
                      preferred_element_type=jnp.float32)

  @pl.when(i == 0)
  def _():
    pool_ref[...] = acc_ref[...]

  pool_ref[...] += p


_tc_layer = pl.pallas_call(
    _tc_body,
    grid=(_NB,),
    in_specs=[
        pl.BlockSpec((_B, _D), lambda i: (i, 0)),
        pl.BlockSpec((_NC, _B, _D), lambda i: (0, i, 0)),
        pl.BlockSpec((_NC, _B, 1), lambda i: (0, i, 0)),
        pl.BlockSpec((_MAXD + 1, 2 * _D, _D), lambda i: (0, 0, 0)),
        pl.BlockSpec((16, _D), lambda i: (0, 0)),
        pl.BlockSpec((_D, _D), lambda i: (0, 0)),
        pl.BlockSpec((_B, 1), lambda i: (i, 0)),
        pl.BlockSpec((_G, _D), lambda i: (0, 0)),
    ],
    out_specs=[
        pl.BlockSpec((_B, _D), lambda i: (i, 0)),
        pl.BlockSpec((_G, _D), lambda i: (0, 0)),
    ],
    out_shape=[
        jax.ShapeDtypeStruct((_N, _D), jnp.float32),
        jax.ShapeDtypeStruct((_G, _D), jnp.float32),
    ],
)


def _mlp_body(acc_ref, mlpwt_ref, mlpb_ref, out_ref):
  out_ref[...] = (jnp.dot(acc_ref[...], mlpwt_ref[...],
                          preferred_element_type=jnp.float32)
                  + mlpb_ref[...])


_tc_mlp = pl.pallas_call(
    _mlp_body,
    out_shape=jax.ShapeDtypeStruct((_G, _D), jnp.float32),
)


@jax.jit
def kernel(x, edge_index, batch, Wl, bl, Wr, lin_W, mlp_W, mlp_b):
  src = edge_index[0]
  dst = edge_index[1]
  z128 = jnp.zeros((_CHUNK, _D), jnp.float32)
  wcat = jnp.concatenate(
      [jnp.transpose(Wl, (0, 1, 3, 2)), jnp.transpose(Wr, (0, 1, 3, 2))],
      axis=2)
  blp = jnp.pad(bl, ((0, 0), (0, 16 - (_MAXD + 1)), (0, 0)))
  linwt = jnp.transpose(lin_W, (0, 2, 1))
  mlpwt = mlp_W.T
  mlpb2 = mlp_b[None, :]
  batch2 = batch[:, None]
  acc = jnp.zeros((_G, _D), jnp.float32)

  eye = jnp.tile(jnp.eye(16, _D, dtype=jnp.float32), (8, 1))
  deg_plane = _sc_deg(dst, eye, z128)
  deg2 = deg_plane[:, :, :16].reshape(_NC, _NDR * 16, 1)[:, :_N]

  def layer(carry, ws):
    xc, accc = carry
    wcat_l, bl_l, linwt_l = ws
    h2 = _sc_agg(xc, src, dst, z128)
    xn, accn = _tc_layer(xc, h2, deg2, wcat_l, bl_l, linwt_l, batch2, accc)
    return (xn, accn), None

  (x, acc), _ = lax.scan(layer, (x, acc), (wcat, blp, linwt))
  return _tc_mlp(acc, mlpwt, mlpb2)
